# baseline (speedup 1.0000x reference)
import functools

import jax
import jax.numpy as jnp
from jax import lax
from jax.experimental import pallas as pl
from jax.experimental.pallas import tpu as pltpu
from jax.experimental.pallas import tpu_sc as plsc

G = 64
GA = 80
L = 128
S = 3
R = S * L
NC = 2
NS = 16
NW = NC * NS
TB = 1024
TC_FRAC = 0.60


def _sc_partial(x, batch, n0, n, d, nsg):
    mesh = plsc.VectorSubcoreMesh(
        core_axis_name="c", subcore_axis_name="s",
        num_cores=NC, num_subcores=NS)
    nfull, nrem = nsg // NW, nsg % NW
    nit = nfull + (1 if nrem else 0)
    pad = n0 + nsg * R - n
    assert pad % 16 == 0 and pad < R and nfull >= 1 and n - n0 >= R
    last_w, last_k = (nsg - 1) % NW, (nsg - 1) // NW

    @functools.partial(
        pl.kernel,
        out_type=jax.ShapeDtypeStruct((NC, GA, d), jnp.float32),
        mesh=mesh,
        scratch_types=[
            pltpu.VMEM((R, d), jnp.float32),
            pltpu.VMEM((R, d), jnp.float32),
            pltpu.VMEM((S, L), jnp.int32),
            pltpu.VMEM((S, L), jnp.int32),
            pltpu.VMEM((GA, d), jnp.float32),
            pltpu.VMEM_SHARED((GA, d), jnp.float32),
            pltpu.SemaphoreType.DMA,
            pltpu.SemaphoreType.DMA,
            pltpu.SemaphoreType.DMA,
            pltpu.SemaphoreType.DMA,
            pltpu.SemaphoreType.DMA,
            pltpu.SemaphoreType.DMA,
        ],
    )
    def k(x_hbm, b_hbm, psum_hbm,
          xbuf0, xbuf1, ibuf0, ibuf1, outv, acc_sh,
          semx0, semx1, semi0, semi1, sems0, sems1):
        c = lax.axis_index("c")
        s = lax.axis_index("s")
        wid = s * NC + c
        xbufs, ibufs = (xbuf0, xbuf1), (ibuf0, ibuf1)
        semx, semi, sems = (semx0, semx1), (semi0, semi1), (sems0, sems1)

        def load_descs(kk):
            b = kk & 1
            t = wid + kk * NW
            base = jnp.minimum(n0 + t * R, n - R)
            descs = [pltpu.make_async_copy(
                x_hbm.at[pl.ds(base, R)], xbufs[b], semx[b])]
            for j in range(S):
                descs.append(pltpu.make_async_copy(
                    b_hbm.at[pl.ds(base + j * L, L)], ibufs[b].at[j], semi[b]))
            return descs

        def issue_loads(kk):
            for dsc in load_descs(kk):
                dsc.start()

        def wait_loads(kk):
            for dsc in load_descs(kk):
                dsc.wait()

        def scatter_descs(kk):
            b = kk & 1
            return [pltpu.make_async_copy(
                        xbufs[b].at[pl.ds(j * L, L)],
                        acc_sh.at[ibufs[b].at[j]], sems[b])
                    for j in range(S)]

        zero16 = jnp.zeros((16,), jnp.float32)
        dummy16 = jnp.full((16,), G, jnp.int32)

        def fix_pad(kk):
            b = kk & 1
            if pad and kk == last_k:
                @pl.when(wid == last_w)
                def _():
                    for e in range(pad // 16):
                        ibufs[b][e // (L // 16),
                                 pl.ds((e % (L // 16)) * 16, 16)] = dummy16

        issue_loads(0)

        @pl.when(s == 0)
        def _():
            def zrow(r_, carry):
                for f in range(d // 16):
                    outv[r_, pl.ds(f * 16, 16)] = zero16
                return carry
            lax.fori_loop(0, GA, zrow, 0)
            pltpu.sync_copy(outv, acc_sh)
        plsc.subcore_barrier()

        for kk in range(nit):
            guarded = nrem and kk == nfull
            if kk >= 1:
                for dsc in scatter_descs(kk - 1):
                    dsc.wait()
            if guarded:
                @pl.when(wid < nrem)
                def _(kk=kk):
                    wait_loads(kk)
                    fix_pad(kk)
                    for dsc in scatter_descs(kk):
                        dsc.start(add=True)
            else:
                wait_loads(kk)
                fix_pad(kk)
                for dsc in scatter_descs(kk):
                    dsc.start(add=True)
                if kk + 1 < nit:
                    if nrem and kk + 1 == nfull:
                        @pl.when(wid < nrem)
                        def _(kk=kk):
                            issue_loads(kk + 1)
                    else:
                        issue_loads(kk + 1)
        if nrem:
            @pl.when(wid < nrem)
            def _():
                for dsc in scatter_descs(nit - 1):
                    dsc.wait()
        else:
            for dsc in scatter_descs(nit - 1):
                dsc.wait()
        plsc.subcore_barrier()

        @pl.when(s == 0)
        def _():
            pltpu.sync_copy(acc_sh, outv)
            pltpu.sync_copy(outv, psum_hbm.at[c])

    return k(x, batch)


def _tc_segsum(x, b2, m, d):
    def body(b_ref, x_ref, o_ref, acc):
        i = pl.program_id(0)

        @pl.when(i == 0)
        def _():
            acc[...] = jnp.zeros_like(acc)

        mask = (lax.broadcasted_iota(jnp.int32, (G, TB), 0)
                == b_ref[0]).astype(jnp.bfloat16)
        acc[...] += jnp.dot(mask, x_ref[...].astype(jnp.bfloat16),
                            preferred_element_type=jnp.float32)

        @pl.when(i == m - 1)
        def _():
            o_ref[...] = acc[...]

    return pl.pallas_call(
        body,
        grid=(m,),
        in_specs=[pl.BlockSpec((1, 1, TB), lambda i: (i, 0, 0)),
                  pl.BlockSpec((TB, d), lambda i: (i, 0))],
        out_specs=pl.BlockSpec((G, d), lambda i: (0, 0)),
        out_shape=jax.ShapeDtypeStruct((G, d), jnp.float32),
        scratch_shapes=[pltpu.VMEM((G, d), jnp.float32)],
    )(b2, x)


def _count_tc(bp):
    def body(b_ref, o_ref):
        b2 = b_ref[...]
        per_lane = [jnp.sum(jnp.where(b2 == g, 1.0, 0.0), axis=0)
                    for g in range(G)]
        o_ref[...] = jnp.sum(jnp.stack(per_lane, axis=0), axis=1,
                             keepdims=True)

    return pl.pallas_call(
        body,
        out_shape=jax.ShapeDtypeStruct((G, 1), jnp.float32),
    )(bp)


def _finalize(psum, tsum, cnt, d):
    def body(ps_ref, ts_ref, c_ref, o_ref):
        sums = ps_ref[0, :G, :] + ps_ref[1, :G, :] + ts_ref[...]
        o_ref[...] = sums / jnp.maximum(c_ref[...], 1.0)

    return pl.pallas_call(
        body,
        out_shape=jax.ShapeDtypeStruct((G, d), jnp.float32),
    )(psum, tsum, cnt)


def kernel(x, edge_index, batch):
    n, d = x.shape
    m = int(n * TC_FRAC) // TB
    n0 = m * TB
    nsg = (n - n0 + R - 1) // R
    ng = (n + L - 1) // L
    bp = jnp.concatenate(
        [batch, jnp.full((ng * L - n,), G, jnp.int32)]).reshape(ng, L)
    b2 = batch[:n0].reshape(m, 1, TB)
    psum = _sc_partial(x, batch, n0, n, d, nsg)
    tsum = _tc_segsum(x, b2, m, d)
    cnt = _count_tc(bp)
    return _finalize(psum, tsum, cnt, d)


# speedup vs baseline: 1.2108x; 1.2108x over previous
import functools

import jax
import jax.numpy as jnp
from jax import lax
from jax.experimental import pallas as pl
from jax.experimental.pallas import tpu as pltpu
from jax.experimental.pallas import tpu_sc as plsc

G = 64
GA = 80
L = 128
S = 3
R = S * L
NC = 2
NS = 16
NW = NC * NS
TB = 1024
TC_FRAC = 0.40


def _sc_partial(x, batch, n0, n, d, nsg):
    mesh = plsc.VectorSubcoreMesh(
        core_axis_name="c", subcore_axis_name="s",
        num_cores=NC, num_subcores=NS)
    nfull, nrem = nsg // NW, nsg % NW
    nit = nfull + (1 if nrem else 0)
    pad = n0 + nsg * R - n
    assert pad % 16 == 0 and pad < R and nfull >= 1 and n - n0 >= R
    last_w, last_k = (nsg - 1) % NW, (nsg - 1) // NW

    @functools.partial(
        pl.kernel,
        out_type=jax.ShapeDtypeStruct((NC, GA, d), jnp.float32),
        mesh=mesh,
        scratch_types=[
            pltpu.VMEM((R, d), jnp.float32),
            pltpu.VMEM((R, d), jnp.float32),
            pltpu.VMEM((S, L), jnp.int32),
            pltpu.VMEM((S, L), jnp.int32),
            pltpu.VMEM((GA, d), jnp.float32),
            pltpu.VMEM_SHARED((GA, d), jnp.float32),
            pltpu.SemaphoreType.DMA,
            pltpu.SemaphoreType.DMA,
            pltpu.SemaphoreType.DMA,
            pltpu.SemaphoreType.DMA,
            pltpu.SemaphoreType.DMA,
            pltpu.SemaphoreType.DMA,
        ],
    )
    def k(x_hbm, b_hbm, psum_hbm,
          xbuf0, xbuf1, ibuf0, ibuf1, outv, acc_sh,
          semx0, semx1, semi0, semi1, sems0, sems1):
        c = lax.axis_index("c")
        s = lax.axis_index("s")
        wid = s * NC + c
        xbufs, ibufs = (xbuf0, xbuf1), (ibuf0, ibuf1)
        semx, semi, sems = (semx0, semx1), (semi0, semi1), (sems0, sems1)

        def load_descs(kk):
            b = kk & 1
            t = wid + kk * NW
            base = jnp.minimum(n0 + t * R, n - R)
            descs = [pltpu.make_async_copy(
                x_hbm.at[pl.ds(base, R)], xbufs[b], semx[b])]
            for j in range(S):
                descs.append(pltpu.make_async_copy(
                    b_hbm.at[pl.ds(base + j * L, L)], ibufs[b].at[j], semi[b]))
            return descs

        def issue_loads(kk):
            for dsc in load_descs(kk):
                dsc.start()

        def wait_loads(kk):
            for dsc in load_descs(kk):
                dsc.wait()

        def scatter_descs(kk):
            b = kk & 1
            return [pltpu.make_async_copy(
                        xbufs[b].at[pl.ds(j * L, L)],
                        acc_sh.at[ibufs[b].at[j]], sems[b])
                    for j in range(S)]

        zero16 = jnp.zeros((16,), jnp.float32)
        dummy16 = jnp.full((16,), G, jnp.int32)

        def fix_pad(kk):
            b = kk & 1
            if pad and kk == last_k:
                @pl.when(wid == last_w)
                def _():
                    for e in range(pad // 16):
                        ibufs[b][e // (L // 16),
                                 pl.ds((e % (L // 16)) * 16, 16)] = dummy16

        issue_loads(0)

        @pl.when(s == 0)
        def _():
            def zrow(r_, carry):
                for f in range(d // 16):
                    outv[r_, pl.ds(f * 16, 16)] = zero16
                return carry
            lax.fori_loop(0, GA, zrow, 0)
            pltpu.sync_copy(outv, acc_sh)
        plsc.subcore_barrier()

        for kk in range(nit):
            guarded = nrem and kk == nfull
            if kk >= 1:
                for dsc in scatter_descs(kk - 1):
                    dsc.wait()
            if guarded:
                @pl.when(wid < nrem)
                def _(kk=kk):
                    wait_loads(kk)
                    fix_pad(kk)
                    for dsc in scatter_descs(kk):
                        dsc.start(add=True)
            else:
                wait_loads(kk)
                fix_pad(kk)
                for dsc in scatter_descs(kk):
                    dsc.start(add=True)
                if kk + 1 < nit:
                    if nrem and kk + 1 == nfull:
                        @pl.when(wid < nrem)
                        def _(kk=kk):
                            issue_loads(kk + 1)
                    else:
                        issue_loads(kk + 1)
        if nrem:
            @pl.when(wid < nrem)
            def _():
                for dsc in scatter_descs(nit - 1):
                    dsc.wait()
        else:
            for dsc in scatter_descs(nit - 1):
                dsc.wait()
        plsc.subcore_barrier()

        @pl.when(s == 0)
        def _():
            pltpu.sync_copy(acc_sh, outv)
            pltpu.sync_copy(outv, psum_hbm.at[c])

    return k(x, batch)


def _tc_segsum(x, b2, m, d):
    def body(b_ref, x_ref, o_ref, acc):
        i = pl.program_id(0)

        @pl.when(i == 0)
        def _():
            acc[...] = jnp.zeros_like(acc)

        mask = (lax.broadcasted_iota(jnp.int32, (G, TB), 0)
                == b_ref[0]).astype(jnp.bfloat16)
        acc[...] += jnp.dot(mask, x_ref[...].astype(jnp.bfloat16),
                            preferred_element_type=jnp.float32)

        @pl.when(i == m - 1)
        def _():
            o_ref[...] = acc[...]

    return pl.pallas_call(
        body,
        grid=(m,),
        in_specs=[pl.BlockSpec((1, 1, TB), lambda i: (i, 0, 0)),
                  pl.BlockSpec((TB, d), lambda i: (i, 0))],
        out_specs=pl.BlockSpec((G, d), lambda i: (0, 0)),
        out_shape=jax.ShapeDtypeStruct((G, d), jnp.float32),
        scratch_shapes=[pltpu.VMEM((G, d), jnp.float32)],
    )(b2, x)


def _count_tc(bp):
    def body(b_ref, o_ref):
        b2 = b_ref[...]
        per_lane = [jnp.sum(jnp.where(b2 == g, 1.0, 0.0), axis=0)
                    for g in range(G)]
        o_ref[...] = jnp.sum(jnp.stack(per_lane, axis=0), axis=1,
                             keepdims=True)

    return pl.pallas_call(
        body,
        out_shape=jax.ShapeDtypeStruct((G, 1), jnp.float32),
    )(bp)


def _finalize(psum, tsum, cnt, d):
    def body(ps_ref, ts_ref, c_ref, o_ref):
        sums = ps_ref[0, :G, :] + ps_ref[1, :G, :] + ts_ref[...]
        o_ref[...] = sums / jnp.maximum(c_ref[...], 1.0)

    return pl.pallas_call(
        body,
        out_shape=jax.ShapeDtypeStruct((G, d), jnp.float32),
    )(psum, tsum, cnt)


def kernel(x, edge_index, batch):
    n, d = x.shape
    m = int(n * TC_FRAC) // TB
    n0 = m * TB
    nsg = (n - n0 + R - 1) // R
    ng = (n + L - 1) // L
    bp = jnp.concatenate(
        [batch, jnp.full((ng * L - n,), G, jnp.int32)]).reshape(ng, L)
    b2 = batch[:n0].reshape(m, 1, TB)
    psum = _sc_partial(x, batch, n0, n, d, nsg)
    tsum = _tc_segsum(x, b2, m, d)
    cnt = _count_tc(bp)
    return _finalize(psum, tsum, cnt, d)


# speedup vs baseline: 1.3005x; 1.0741x over previous
import functools

import jax
import jax.numpy as jnp
from jax import lax
from jax.experimental import pallas as pl
from jax.experimental.pallas import tpu as pltpu
from jax.experimental.pallas import tpu_sc as plsc

G = 64
GA = 80
L = 128
S = 3
R = S * L
NC = 2
NS = 16
NW = NC * NS
TB = 1024
TC_FRAC = 0.35


def _sc_partial(x, batch, n0, n, d, nsg):
    mesh = plsc.VectorSubcoreMesh(
        core_axis_name="c", subcore_axis_name="s",
        num_cores=NC, num_subcores=NS)
    nfull, nrem = nsg // NW, nsg % NW
    nit = nfull + (1 if nrem else 0)
    pad = n0 + nsg * R - n
    assert pad % 16 == 0 and pad < R and nfull >= 1 and n - n0 >= R
    last_w, last_k = (nsg - 1) % NW, (nsg - 1) // NW

    @functools.partial(
        pl.kernel,
        out_type=jax.ShapeDtypeStruct((NC, GA, d), jnp.float32),
        mesh=mesh,
        scratch_types=[
            pltpu.VMEM((R, d), jnp.float32),
            pltpu.VMEM((R, d), jnp.float32),
            pltpu.VMEM((S, L), jnp.int32),
            pltpu.VMEM((S, L), jnp.int32),
            pltpu.VMEM((GA, d), jnp.float32),
            pltpu.VMEM_SHARED((GA, d), jnp.float32),
            pltpu.SemaphoreType.DMA,
            pltpu.SemaphoreType.DMA,
            pltpu.SemaphoreType.DMA,
            pltpu.SemaphoreType.DMA,
            pltpu.SemaphoreType.DMA,
            pltpu.SemaphoreType.DMA,
        ],
    )
    def k(x_hbm, b_hbm, psum_hbm,
          xbuf0, xbuf1, ibuf0, ibuf1, outv, acc_sh,
          semx0, semx1, semi0, semi1, sems0, sems1):
        c = lax.axis_index("c")
        s = lax.axis_index("s")
        wid = s * NC + c
        xbufs, ibufs = (xbuf0, xbuf1), (ibuf0, ibuf1)
        semx, semi, sems = (semx0, semx1), (semi0, semi1), (sems0, sems1)

        def load_descs(kk):
            b = kk & 1
            t = wid + kk * NW
            base = jnp.minimum(n0 + t * R, n - R)
            descs = [pltpu.make_async_copy(
                x_hbm.at[pl.ds(base, R)], xbufs[b], semx[b])]
            for j in range(S):
                descs.append(pltpu.make_async_copy(
                    b_hbm.at[pl.ds(base + j * L, L)], ibufs[b].at[j], semi[b]))
            return descs

        def issue_loads(kk):
            for dsc in load_descs(kk):
                dsc.start()

        def wait_loads(kk):
            for dsc in load_descs(kk):
                dsc.wait()

        def scatter_descs(kk):
            b = kk & 1
            return [pltpu.make_async_copy(
                        xbufs[b].at[pl.ds(j * L, L)],
                        acc_sh.at[ibufs[b].at[j]], sems[b])
                    for j in range(S)]

        zero16 = jnp.zeros((16,), jnp.float32)
        dummy16 = jnp.full((16,), G, jnp.int32)

        def fix_pad(kk):
            b = kk & 1
            if pad and kk == last_k:
                @pl.when(wid == last_w)
                def _():
                    for e in range(pad // 16):
                        ibufs[b][e // (L // 16),
                                 pl.ds((e % (L // 16)) * 16, 16)] = dummy16

        issue_loads(0)

        @pl.when(s == 0)
        def _():
            def zrow(r_, carry):
                for f in range(d // 16):
                    outv[r_, pl.ds(f * 16, 16)] = zero16
                return carry
            lax.fori_loop(0, GA, zrow, 0)
            pltpu.sync_copy(outv, acc_sh)
        plsc.subcore_barrier()

        for kk in range(nit):
            guarded = nrem and kk == nfull
            if kk >= 1:
                for dsc in scatter_descs(kk - 1):
                    dsc.wait()
            if guarded:
                @pl.when(wid < nrem)
                def _(kk=kk):
                    wait_loads(kk)
                    fix_pad(kk)
                    for dsc in scatter_descs(kk):
                        dsc.start(add=True)
            else:
                wait_loads(kk)
                fix_pad(kk)
                for dsc in scatter_descs(kk):
                    dsc.start(add=True)
                if kk + 1 < nit:
                    if nrem and kk + 1 == nfull:
                        @pl.when(wid < nrem)
                        def _(kk=kk):
                            issue_loads(kk + 1)
                    else:
                        issue_loads(kk + 1)
        if nrem:
            @pl.when(wid < nrem)
            def _():
                for dsc in scatter_descs(nit - 1):
                    dsc.wait()
        else:
            for dsc in scatter_descs(nit - 1):
                dsc.wait()
        plsc.subcore_barrier()

        @pl.when(s == 0)
        def _():
            pltpu.sync_copy(acc_sh, outv)
            pltpu.sync_copy(outv, psum_hbm.at[c])

    return k(x, batch)


def _tc_segsum(x, b2, m, d):
    def body(b_ref, x_ref, o_ref, acc):
        i = pl.program_id(0)

        @pl.when(i == 0)
        def _():
            acc[...] = jnp.zeros_like(acc)

        mask = (lax.broadcasted_iota(jnp.int32, (G, TB), 0)
                == b_ref[0]).astype(jnp.bfloat16)
        acc[...] += jnp.dot(mask, x_ref[...].astype(jnp.bfloat16),
                            preferred_element_type=jnp.float32)

        @pl.when(i == m - 1)
        def _():
            o_ref[...] = acc[...]

    return pl.pallas_call(
        body,
        grid=(m,),
        in_specs=[pl.BlockSpec((1, 1, TB), lambda i: (i, 0, 0)),
                  pl.BlockSpec((TB, d), lambda i: (i, 0))],
        out_specs=pl.BlockSpec((G, d), lambda i: (0, 0)),
        out_shape=jax.ShapeDtypeStruct((G, d), jnp.float32),
        scratch_shapes=[pltpu.VMEM((G, d), jnp.float32)],
    )(b2, x)


def _count_tc(bp):
    def body(b_ref, o_ref):
        b2 = b_ref[...]
        per_lane = [jnp.sum(jnp.where(b2 == g, 1.0, 0.0), axis=0)
                    for g in range(G)]
        o_ref[...] = jnp.sum(jnp.stack(per_lane, axis=0), axis=1,
                             keepdims=True)

    return pl.pallas_call(
        body,
        out_shape=jax.ShapeDtypeStruct((G, 1), jnp.float32),
    )(bp)


def _finalize(psum, tsum, cnt, d):
    def body(ps_ref, ts_ref, c_ref, o_ref):
        sums = ps_ref[0, :G, :] + ps_ref[1, :G, :] + ts_ref[...]
        o_ref[...] = sums / jnp.maximum(c_ref[...], 1.0)

    return pl.pallas_call(
        body,
        out_shape=jax.ShapeDtypeStruct((G, d), jnp.float32),
    )(psum, tsum, cnt)


def kernel(x, edge_index, batch):
    n, d = x.shape
    m = int(n * TC_FRAC) // TB
    n0 = m * TB
    nsg = (n - n0 + R - 1) // R
    ng = (n + L - 1) // L
    bp = jnp.concatenate(
        [batch, jnp.full((ng * L - n,), G, jnp.int32)]).reshape(ng, L)
    b2 = batch[:n0].reshape(m, 1, TB)
    psum = _sc_partial(x, batch, n0, n, d, nsg)
    tsum = _tc_segsum(x, b2, m, d)
    cnt = _count_tc(bp)
    return _finalize(psum, tsum, cnt, d)
